# Initial kernel scaffold; baseline (speedup 1.0000x reference)
import jax
import jax.numpy as jnp
from jax.experimental import pallas as pl


def kernel(x, Wg, bg, Wx, Wdt, bdt, A_log, Dexp, g1, b1, g2, b2):
    raise NotImplementedError("write your pallas kernel here")


# speedup vs baseline: 15.3103x; 15.3103x over previous
import functools

import jax
import jax.numpy as jnp
from jax.experimental import pallas as pl
from jax.experimental.pallas import tpu as pltpu

DIM = 384
E = 8
TOPK = 2
H = 7
W = 7
B = 16
DSTATE = 16
DTRANK = 48
L = H * W
CAP = 20.0
EPS = 1e-6
_HIGHEST = jax.lax.Precision.HIGHEST


def _softplus(x):
    return jnp.where(x > 0.0, x, 0.0) + jnp.log1p(jnp.exp(-jnp.abs(x)))


def _ln_rows(x, g, b, eps=1e-5):
    m = jnp.mean(x, axis=-1, keepdims=True)
    v = jnp.mean((x - m) ** 2, axis=-1, keepdims=True)
    return (x - m) * jax.lax.rsqrt(v + eps) * g + b


def _top1(vals, iota):
    m = jnp.max(vals, axis=1, keepdims=True)
    idx = jnp.min(jnp.where(vals == m, iota, E), axis=1, keepdims=True)
    return m, idx


def _gate_body(x_ref, Wg_ref, bg_ref, sval_ref, sidx_ref, aux_ref):
    xs = x_ref[...]
    xf = jnp.mean(xs, axis=(1, 2))
    logits = jax.lax.dot_general(xf, Wg_ref[...], (((1,), (0,)), ((), ())),
                                 precision=_HIGHEST) + bg_ref[...]
    z = logits - jnp.max(logits, axis=1, keepdims=True)
    ez = jnp.exp(z)
    raw = ez / jnp.sum(ez, axis=1, keepdims=True)

    iota = jax.lax.broadcasted_iota(jnp.int32, (B, E), 1)
    _, i1 = _top1(raw, iota)
    mask1 = iota == i1
    _, i2 = _top1(jnp.where(mask1, -1.0, raw), iota)
    mask = jnp.logical_or(mask1, iota == i2).astype(jnp.float32)

    masked = raw * mask
    denom = jnp.sum(masked, axis=0, keepdims=True) + EPS
    gs = masked / denom * CAP

    importance = jnp.mean(raw, axis=0, keepdims=True)
    load = jnp.mean(mask, axis=0, keepdims=True)
    aux_ref[...] = 0.01 * jnp.mean((load - importance) ** 2,
                                   axis=(0, 1), keepdims=True)

    g1v, gi1 = _top1(gs, iota)
    g2v, gi2 = _top1(jnp.where(iota == gi1, -1.0, gs), iota)
    sval_ref[...] = jnp.concatenate([g1v, g2v], axis=1)
    sidx_ref[...] = jnp.concatenate([gi1, gi2], axis=1).astype(jnp.int32)


def _expert_body(sidx_ref, sval_ref, x_ref, Wx_ref, Wdt_ref, bdt_ref,
                 Alog_ref, Dexp_ref, g1_ref, b1_ref, g2_ref, b2_ref,
                 out_ref, da_ref, dbu_ref, hs_ref):
    b = pl.program_id(0)
    k = pl.program_id(1)

    pio = jax.lax.broadcasted_iota(jnp.int32, (L, L), 0)
    qio = jax.lax.broadcasted_iota(jnp.int32, (L, L), 1)
    T = jnp.logical_and(pio // W == qio % H, pio % W == qio // H).astype(jnp.float32)

    seq_h = x_ref[0].reshape(L, DIM)
    seq_v = jax.lax.dot_general(T, seq_h, (((1,), (0,)), ((), ())),
                                precision=_HIGHEST)
    seq2 = jnp.concatenate([seq_h, seq_v], axis=0)

    x_dbl = jax.lax.dot_general(seq2, Wx_ref[0], (((1,), (0,)), ((), ())),
                                precision=_HIGHEST)
    dt_raw = x_dbl[:, :DTRANK]
    delta2 = _softplus(
        jax.lax.dot_general(dt_raw, Wdt_ref[0], (((1,), (0,)), ((), ())),
                            precision=_HIGHEST) + bdt_ref[0])
    Bm2 = x_dbl[:, DTRANK:DTRANK + DSTATE]
    Cm2 = x_dbl[:, DTRANK + DSTATE:]

    AT = -jnp.exp(Alog_ref[0]).T

    dd = jnp.stack([delta2[:L], delta2[L:]], axis=1)
    uu = jnp.stack([seq_h, seq_v], axis=1)
    du = dd * uu
    BBr = jnp.stack([Bm2[:L], Bm2[L:]], axis=1)[:, :, :, None]
    CCr = jnp.stack([Cm2[:L], Cm2[L:]], axis=1)[:, :, :, None]

    for c in range(7):
        s = slice(7 * c, 7 * c + 7)
        da_ref[s] = jnp.exp(dd[s][:, :, None, :] * AT[None, None])
        dbu_ref[s] = du[s][:, :, None, :] * BBr[s]

    def step(t, carry):
        hf, hb = carry
        hf = da_ref[t] * hf + dbu_ref[t]
        r = L - 1 - t
        hb = da_ref[r] * hb + dbu_ref[r]
        hs_ref[t, 0:2, :, :] = hf
        hs_ref[t, 2:4, :, :] = hb
        return hf, hb

    h0 = jnp.zeros((2, DSTATE, DIM), jnp.float32)
    jax.lax.fori_loop(0, L, step, (h0, h0))

    yfs, ybs = [], []
    for c in range(7):
        s = slice(7 * c, 7 * c + 7)
        yfs.append(jnp.sum(hs_ref[s, 0:2] * CCr[s], axis=2))
        Hb = jnp.stack([hs_ref[L - 1 - (7 * c + j), 2:4] for j in range(7)],
                       axis=0)
        ybs.append(jnp.sum(Hb * CCr[s], axis=2))
    y_f = jnp.concatenate(yfs, axis=0)
    y_b = jnp.concatenate(ybs, axis=0)

    y_hsum = y_f[:, 0] + y_b[:, 0]
    y_vsum = y_f[:, 1] + y_b[:, 1]
    y_v_un = jax.lax.dot_general(T, y_vsum, (((1,), (0,)), ((), ())),
                                 precision=_HIGHEST)
    y = y_hsum + y_v_un + 4.0 * seq_h * Dexp_ref[0]

    y = _ln_rows(y, g1_ref[0], b1_ref[0])
    pooled = jnp.mean(y, axis=0, keepdims=True)
    outv = _ln_rows(pooled, g2_ref[0], b2_ref[0])

    contrib = sval_ref[b, k] * outv

    @pl.when(k == 0)
    def _():
        out_ref[0] = contrib

    @pl.when(k == 1)
    def _():
        out_ref[0] += contrib


@jax.jit
def kernel(x, Wg, bg, Wx, Wdt, bdt, A_log, Dexp, g1, b1, g2, b2):
    sval, sidx, aux = pl.pallas_call(
        _gate_body,
        out_shape=[
            jax.ShapeDtypeStruct((B, TOPK), jnp.float32),
            jax.ShapeDtypeStruct((B, TOPK), jnp.int32),
            jax.ShapeDtypeStruct((1, 1), jnp.float32),
        ],
    )(x, Wg, bg.reshape(1, E))

    def e_map3(b, k, sidx_ref, sval_ref):
        return (sidx_ref[b, k], 0, 0)

    grid_spec = pltpu.PrefetchScalarGridSpec(
        num_scalar_prefetch=2,
        grid=(B, TOPK),
        in_specs=[
            pl.BlockSpec((1, H, W, DIM), lambda b, k, si, sv: (b, 0, 0, 0)),
            pl.BlockSpec((1, DIM, DTRANK + 2 * DSTATE), e_map3),
            pl.BlockSpec((1, DTRANK, DIM), e_map3),
            pl.BlockSpec((1, 1, DIM), e_map3),
            pl.BlockSpec((1, DIM, DSTATE), e_map3),
            pl.BlockSpec((1, 1, DIM), e_map3),
            pl.BlockSpec((1, 1, DIM), e_map3),
            pl.BlockSpec((1, 1, DIM), e_map3),
            pl.BlockSpec((1, 1, DIM), e_map3),
            pl.BlockSpec((1, 1, DIM), e_map3),
        ],
        out_specs=pl.BlockSpec((1, 1, DIM), lambda b, k, si, sv: (b, 0, 0)),
        scratch_shapes=[
            pltpu.VMEM((L, 2, DSTATE, DIM), jnp.float32),
            pltpu.VMEM((L, 2, DSTATE, DIM), jnp.float32),
            pltpu.VMEM((L, 4, DSTATE, DIM), jnp.float32),
        ],
    )

    r3 = lambda a: a.reshape(E, 1, DIM)
    mixed = pl.pallas_call(
        _expert_body,
        grid_spec=grid_spec,
        out_shape=jax.ShapeDtypeStruct((B, 1, DIM), jnp.float32),
        compiler_params=pltpu.CompilerParams(
            dimension_semantics=("parallel", "arbitrary"),
        ),
    )(sidx, sval, x, Wx, Wdt, r3(bdt), A_log, r3(Dexp),
      r3(g1), r3(b1), r3(g2), r3(b2))

    return mixed.reshape(B, DIM), aux[0, 0]


# speedup vs baseline: 17.3939x; 1.1361x over previous
import functools

import jax
import jax.numpy as jnp
from jax.experimental import pallas as pl
from jax.experimental.pallas import tpu as pltpu

DIM = 384
E = 8
TOPK = 2
H = 7
W = 7
B = 16
DSTATE = 16
DTRANK = 48
L = H * W
CAP = 20.0
EPS = 1e-6
_HIGHEST = jax.lax.Precision.HIGHEST


def _softplus(x):
    return jnp.where(x > 0.0, x, 0.0) + jnp.log1p(jnp.exp(-jnp.abs(x)))


def _ln_rows(x, g, b, eps=1e-5):
    m = jnp.mean(x, axis=-1, keepdims=True)
    v = jnp.mean((x - m) ** 2, axis=-1, keepdims=True)
    return (x - m) * jax.lax.rsqrt(v + eps) * g + b


def _top1(vals, iota):
    m = jnp.max(vals, axis=1, keepdims=True)
    idx = jnp.min(jnp.where(vals == m, iota, E), axis=1, keepdims=True)
    return m, idx


def _gate_body(x_ref, Wg_ref, bg_ref, sval_ref, sidx_ref, aux_ref):
    xs = x_ref[...]
    xf = jnp.mean(xs, axis=(1, 2))
    logits = jax.lax.dot_general(xf, Wg_ref[...], (((1,), (0,)), ((), ())),
                                 precision=_HIGHEST) + bg_ref[...]
    z = logits - jnp.max(logits, axis=1, keepdims=True)
    ez = jnp.exp(z)
    raw = ez / jnp.sum(ez, axis=1, keepdims=True)

    iota = jax.lax.broadcasted_iota(jnp.int32, (B, E), 1)
    _, i1 = _top1(raw, iota)
    mask1 = iota == i1
    _, i2 = _top1(jnp.where(mask1, -1.0, raw), iota)
    mask = jnp.logical_or(mask1, iota == i2).astype(jnp.float32)

    masked = raw * mask
    denom = jnp.sum(masked, axis=0, keepdims=True) + EPS
    gs = masked / denom * CAP

    importance = jnp.mean(raw, axis=0, keepdims=True)
    load = jnp.mean(mask, axis=0, keepdims=True)
    aux_ref[...] = 0.01 * jnp.mean((load - importance) ** 2,
                                   axis=(0, 1), keepdims=True)

    g1v, gi1 = _top1(gs, iota)
    g2v, gi2 = _top1(jnp.where(iota == gi1, -1.0, gs), iota)
    sval_ref[...] = jnp.concatenate([g1v, g2v], axis=1)
    sidx_ref[...] = jnp.concatenate([gi1, gi2], axis=1).astype(jnp.int32)


def _expert_body(sidx_ref, sval_ref, x_ref, Wx_ref, Wdt_ref, bdt_ref,
                 Alog_ref, Dexp_ref, g1_ref, b1_ref, g2_ref, b2_ref,
                 out_ref, da_ref, dbu_ref, hs_ref):
    b = pl.program_id(0)
    k = pl.program_id(1)

    pio = jax.lax.broadcasted_iota(jnp.int32, (L, L), 0)
    qio = jax.lax.broadcasted_iota(jnp.int32, (L, L), 1)
    T = jnp.logical_and(pio // W == qio % H, pio % W == qio // H).astype(jnp.float32)

    seq_h = x_ref[0].reshape(L, DIM)
    seq_v = jax.lax.dot_general(T, seq_h, (((1,), (0,)), ((), ())),
                                precision=_HIGHEST)
    seq2 = jnp.concatenate([seq_h, seq_v], axis=0)

    x_dbl = jax.lax.dot_general(seq2, Wx_ref[0], (((1,), (0,)), ((), ())),
                                precision=_HIGHEST)
    dt_raw = x_dbl[:, :DTRANK]
    delta2 = _softplus(
        jax.lax.dot_general(dt_raw, Wdt_ref[0], (((1,), (0,)), ((), ())),
                            precision=_HIGHEST) + bdt_ref[0])
    Bm2 = x_dbl[:, DTRANK:DTRANK + DSTATE]
    Cm2 = x_dbl[:, DTRANK + DSTATE:]

    AT = -jnp.exp(Alog_ref[0]).T

    dd = jnp.stack([delta2[:L], delta2[L:]], axis=1)
    uu = jnp.stack([seq_h, seq_v], axis=1)
    du = dd * uu
    BBr = jnp.stack([Bm2[:L], Bm2[L:]], axis=1)[:, :, :, None]
    CCr = jnp.stack([Cm2[:L], Cm2[L:]], axis=1)[:, :, :, None]

    for c in range(7):
        s = slice(7 * c, 7 * c + 7)
        da_ref[s] = jnp.exp(dd[s][:, :, None, :] * AT[None, None])
        dbu_ref[s] = du[s][:, :, None, :] * BBr[s]

    hf = jnp.zeros((2, DSTATE, DIM), jnp.float32)
    hb = jnp.zeros((2, DSTATE, DIM), jnp.float32)
    for t in range(L):
        r = L - 1 - t
        hf = da_ref[t] * hf + dbu_ref[t]
        hb = da_ref[r] * hb + dbu_ref[r]
        hs_ref[t, 0:2, :, :] = hf
        hs_ref[r, 2:4, :, :] = hb

    ys = []
    for c in range(7):
        s = slice(7 * c, 7 * c + 7)
        hsum = hs_ref[s, 0:2] + hs_ref[s, 2:4]
        ys.append(jnp.sum(hsum * CCr[s], axis=2))
    y_fb = jnp.concatenate(ys, axis=0)

    y_hsum = y_fb[:, 0]
    y_vsum = y_fb[:, 1]
    y_v_un = jax.lax.dot_general(T, y_vsum, (((1,), (0,)), ((), ())),
                                 precision=_HIGHEST)
    y = y_hsum + y_v_un + 4.0 * seq_h * Dexp_ref[0]

    y = _ln_rows(y, g1_ref[0], b1_ref[0])
    pooled = jnp.mean(y, axis=0, keepdims=True)
    outv = _ln_rows(pooled, g2_ref[0], b2_ref[0])

    contrib = sval_ref[b, k] * outv

    @pl.when(k == 0)
    def _():
        out_ref[0] = contrib

    @pl.when(k == 1)
    def _():
        out_ref[0] += contrib


@jax.jit
def kernel(x, Wg, bg, Wx, Wdt, bdt, A_log, Dexp, g1, b1, g2, b2):
    sval, sidx, aux = pl.pallas_call(
        _gate_body,
        out_shape=[
            jax.ShapeDtypeStruct((B, TOPK), jnp.float32),
            jax.ShapeDtypeStruct((B, TOPK), jnp.int32),
            jax.ShapeDtypeStruct((1, 1), jnp.float32),
        ],
    )(x, Wg, bg.reshape(1, E))

    def e_map3(b, k, sidx_ref, sval_ref):
        return (sidx_ref[b, k], 0, 0)

    grid_spec = pltpu.PrefetchScalarGridSpec(
        num_scalar_prefetch=2,
        grid=(B, TOPK),
        in_specs=[
            pl.BlockSpec((1, H, W, DIM), lambda b, k, si, sv: (b, 0, 0, 0)),
            pl.BlockSpec((1, DIM, DTRANK + 2 * DSTATE), e_map3),
            pl.BlockSpec((1, DTRANK, DIM), e_map3),
            pl.BlockSpec((1, 1, DIM), e_map3),
            pl.BlockSpec((1, DIM, DSTATE), e_map3),
            pl.BlockSpec((1, 1, DIM), e_map3),
            pl.BlockSpec((1, 1, DIM), e_map3),
            pl.BlockSpec((1, 1, DIM), e_map3),
            pl.BlockSpec((1, 1, DIM), e_map3),
            pl.BlockSpec((1, 1, DIM), e_map3),
        ],
        out_specs=pl.BlockSpec((1, 1, DIM), lambda b, k, si, sv: (b, 0, 0)),
        scratch_shapes=[
            pltpu.VMEM((L, 2, DSTATE, DIM), jnp.float32),
            pltpu.VMEM((L, 2, DSTATE, DIM), jnp.float32),
            pltpu.VMEM((L, 4, DSTATE, DIM), jnp.float32),
        ],
    )

    r3 = lambda a: a.reshape(E, 1, DIM)
    mixed = pl.pallas_call(
        _expert_body,
        grid_spec=grid_spec,
        out_shape=jax.ShapeDtypeStruct((B, 1, DIM), jnp.float32),
        compiler_params=pltpu.CompilerParams(
            dimension_semantics=("parallel", "arbitrary"),
        ),
    )(sidx, sval, x, Wx, Wdt, r3(bdt), A_log, r3(Dexp),
      r3(g1), r3(b1), r3(g2), r3(b2))

    return mixed.reshape(B, DIM), aux[0, 0]


# speedup vs baseline: 22.2200x; 1.2775x over previous
import jax
import jax.numpy as jnp
from jax.experimental import pallas as pl
from jax.experimental.pallas import tpu as pltpu

DIM = 384
E = 8
TOPK = 2
H = 7
W = 7
B = 16
DSTATE = 16
DTRANK = 48
L = H * W
NXP = DTRANK + 2 * DSTATE
CAP = 20.0
EPS = 1e-6
_HIGHEST = jax.lax.Precision.HIGHEST


def _softplus(x):
    return jnp.where(x > 0.0, x, 0.0) + jnp.log1p(jnp.exp(-jnp.abs(x)))


def _ln_rows(x, g, b, eps=1e-5):
    m = jnp.mean(x, axis=-1, keepdims=True)
    v = jnp.mean((x - m) ** 2, axis=-1, keepdims=True)
    return (x - m) * jax.lax.rsqrt(v + eps) * g + b


def _top1(vals, iota):
    m = jnp.max(vals, axis=1, keepdims=True)
    idx = jnp.min(jnp.where(vals == m, iota, E), axis=1, keepdims=True)
    return m, idx


def _gate_body(x_ref, Wg_ref, bg_ref, sval_ref, sidx_ref, aux_ref):
    xs = x_ref[...]
    xf = jnp.mean(xs, axis=(1, 2))
    logits = jax.lax.dot_general(xf, Wg_ref[...], (((1,), (0,)), ((), ())),
                                 precision=_HIGHEST) + bg_ref[...]
    z = logits - jnp.max(logits, axis=1, keepdims=True)
    ez = jnp.exp(z)
    raw = ez / jnp.sum(ez, axis=1, keepdims=True)

    iota = jax.lax.broadcasted_iota(jnp.int32, (B, E), 1)
    _, i1 = _top1(raw, iota)
    mask1 = iota == i1
    _, i2 = _top1(jnp.where(mask1, -1.0, raw), iota)
    mask = jnp.logical_or(mask1, iota == i2).astype(jnp.float32)

    masked = raw * mask
    denom = jnp.sum(masked, axis=0, keepdims=True) + EPS
    gs = masked / denom * CAP

    importance = jnp.mean(raw, axis=0, keepdims=True)
    load = jnp.mean(mask, axis=0, keepdims=True)
    aux_ref[...] = 0.01 * jnp.mean((load - importance) ** 2,
                                   axis=(0, 1), keepdims=True)

    g1v, gi1 = _top1(gs, iota)
    g2v, gi2 = _top1(jnp.where(iota == gi1, -1.0, gs), iota)
    sval_ref[...] = jnp.concatenate([g1v, g2v], axis=1)
    sidx_ref[...] = jnp.concatenate([gi1, gi2], axis=1).astype(jnp.int32)


def _expert_body(sidx_ref, sval_ref, x_ref,
                 Wx0_ref, Wdt0_ref, bdt0_ref, Alog0_ref, Dexp0_ref,
                 g10_ref, b10_ref, g20_ref, b20_ref,
                 Wx1_ref, Wdt1_ref, bdt1_ref, Alog1_ref, Dexp1_ref,
                 g11_ref, b11_ref, g21_ref, b21_ref,
                 out_ref, da_ref, dbu_ref, hsf_ref, hsb_ref):
    b = pl.program_id(0)
    Wx = (Wx0_ref, Wx1_ref)
    Wdt = (Wdt0_ref, Wdt1_ref)
    bdt = (bdt0_ref, bdt1_ref)
    Alog = (Alog0_ref, Alog1_ref)
    Dexp = (Dexp0_ref, Dexp1_ref)
    g1 = (g10_ref, g11_ref)
    b1 = (b10_ref, b11_ref)
    g2 = (g20_ref, g21_ref)
    b2 = (b20_ref, b21_ref)

    pio = jax.lax.broadcasted_iota(jnp.int32, (L, L), 0)
    qio = jax.lax.broadcasted_iota(jnp.int32, (L, L), 1)
    T = jnp.logical_and(pio // W == qio % H, pio % W == qio // H).astype(jnp.float32)

    seq_h = x_ref[0].reshape(L, DIM)
    seq_v = jax.lax.dot_general(T, seq_h, (((1,), (0,)), ((), ())),
                                precision=_HIGHEST)
    seq2 = jnp.concatenate([seq_h, seq_v], axis=0)

    Wcat = jnp.concatenate([Wx[0][0], Wx[1][0]], axis=1)
    xd2 = jax.lax.dot_general(seq2, Wcat, (((1,), (0,)), ((), ())),
                              precision=_HIGHEST)

    Cms = []
    for j in range(2):
        xd = xd2[:, j * NXP:(j + 1) * NXP]
        delta2 = _softplus(
            jax.lax.dot_general(xd[:, :DTRANK], Wdt[j][0],
                                (((1,), (0,)), ((), ())),
                                precision=_HIGHEST) + bdt[j][0])
        Bm2 = xd[:, DTRANK:DTRANK + DSTATE]
        Cms.append(xd[:, DTRANK + DSTATE:])
        AT = -jnp.exp(Alog[j][0]).T
        du2 = delta2 * seq2
        for o in range(2):
            Br = Bm2[o * L:(o + 1) * L][:, :, None]
            for c in range(7):
                s = slice(7 * c, 7 * c + 7)
                g = slice(o * L + 7 * c, o * L + 7 * c + 7)
                da_ref[j, o, s] = jnp.exp(delta2[g][:, None, :] * AT[None])
                dbu_ref[j, o, s] = du2[g][:, None, :] * Br[s]

    hf = [[jnp.zeros((DSTATE, DIM), jnp.float32) for _ in range(2)]
          for _ in range(2)]
    hb = [[jnp.zeros((DSTATE, DIM), jnp.float32) for _ in range(2)]
          for _ in range(2)]
    for t in range(L):
        r = L - 1 - t
        for j in range(2):
            for o in range(2):
                hf[j][o] = da_ref[j, o, t] * hf[j][o] + dbu_ref[j, o, t]
                hb[j][o] = da_ref[j, o, r] * hb[j][o] + dbu_ref[j, o, r]
                hsf_ref[j, o, t] = hf[j][o]
                hsb_ref[j, o, r] = hb[j][o]

    outsum = None
    yvs = []
    yhs = []
    for j in range(2):
        Cr = Cms[j][:, :, None]
        yos = []
        for o in range(2):
            chunks = []
            for c in range(7):
                s = slice(7 * c, 7 * c + 7)
                hsum = hsf_ref[j, o, s] + hsb_ref[j, o, s]
                chunks.append(jnp.sum(hsum * Cr[o * L + 7 * c:
                                                o * L + 7 * c + 7], axis=1))
            yos.append(jnp.concatenate(chunks, axis=0))
        yhs.append(yos[0])
        yvs.append(yos[1])

    yv_cat = jnp.concatenate(yvs, axis=1)
    yv_un = jax.lax.dot_general(T, yv_cat, (((1,), (0,)), ((), ())),
                                precision=_HIGHEST)

    for j in range(2):
        y = yhs[j] + yv_un[:, j * DIM:(j + 1) * DIM] \
            + 4.0 * seq_h * Dexp[j][0]
        y = _ln_rows(y, g1[j][0], b1[j][0])
        pooled = jnp.mean(y, axis=0, keepdims=True)
        outv = _ln_rows(pooled, g2[j][0], b2[j][0])
        contrib = sval_ref[b, j] * outv
        outsum = contrib if outsum is None else outsum + contrib

    out_ref[0] = outsum


@jax.jit
def kernel(x, Wg, bg, Wx, Wdt, bdt, A_log, Dexp, g1, b1, g2, b2):
    sval, sidx, aux = pl.pallas_call(
        _gate_body,
        out_shape=[
            jax.ShapeDtypeStruct((B, TOPK), jnp.float32),
            jax.ShapeDtypeStruct((B, TOPK), jnp.int32),
            jax.ShapeDtypeStruct((1, 1), jnp.float32),
        ],
    )(x, Wg, bg.reshape(1, E))

    def expert_specs(j):
        def em3(b, si, sv, _j=j):
            return (si[b, _j], 0, 0)
        return [
            pl.BlockSpec((1, DIM, NXP), em3),
            pl.BlockSpec((1, DTRANK, DIM), em3),
            pl.BlockSpec((1, 1, DIM), em3),
            pl.BlockSpec((1, DIM, DSTATE), em3),
            pl.BlockSpec((1, 1, DIM), em3),
            pl.BlockSpec((1, 1, DIM), em3),
            pl.BlockSpec((1, 1, DIM), em3),
            pl.BlockSpec((1, 1, DIM), em3),
            pl.BlockSpec((1, 1, DIM), em3),
        ]

    grid_spec = pltpu.PrefetchScalarGridSpec(
        num_scalar_prefetch=2,
        grid=(B,),
        in_specs=(
            [pl.BlockSpec((1, H, W, DIM), lambda b, si, sv: (b, 0, 0, 0))]
            + expert_specs(0) + expert_specs(1)
        ),
        out_specs=pl.BlockSpec((1, 1, DIM), lambda b, si, sv: (b, 0, 0)),
        scratch_shapes=[
            pltpu.VMEM((2, 2, L, DSTATE, DIM), jnp.float32),
            pltpu.VMEM((2, 2, L, DSTATE, DIM), jnp.float32),
            pltpu.VMEM((2, 2, L, DSTATE, DIM), jnp.float32),
            pltpu.VMEM((2, 2, L, DSTATE, DIM), jnp.float32),
        ],
    )

    r3 = lambda a: a.reshape(E, 1, DIM)
    eargs = (Wx, Wdt, r3(bdt), A_log, r3(Dexp), r3(g1), r3(b1), r3(g2), r3(b2))
    mixed = pl.pallas_call(
        _expert_body,
        grid_spec=grid_spec,
        out_shape=jax.ShapeDtypeStruct((B, 1, DIM), jnp.float32),
        compiler_params=pltpu.CompilerParams(
            dimension_semantics=("parallel",),
        ),
    )(sidx, sval, x, *eargs, *eargs)

    return mixed.reshape(B, DIM), aux[0, 0]


# speedup vs baseline: 22.2502x; 1.0014x over previous
import jax
import jax.numpy as jnp
from jax.experimental import pallas as pl
from jax.experimental.pallas import tpu as pltpu

DIM = 384
E = 8
TOPK = 2
H = 7
W = 7
B = 16
DSTATE = 16
DTRANK = 48
L = H * W
NXP = DTRANK + 2 * DSTATE
CAP = 20.0
EPS = 1e-6
_HIGHEST = jax.lax.Precision.HIGHEST


def _softplus(x):
    return jnp.where(x > 0.0, x, 0.0) + jnp.log1p(jnp.exp(-jnp.abs(x)))


def _ln_rows(x, g, b, eps=1e-5):
    m = jnp.mean(x, axis=-1, keepdims=True)
    v = jnp.mean((x - m) ** 2, axis=-1, keepdims=True)
    return (x - m) * jax.lax.rsqrt(v + eps) * g + b


def _top1(vals, iota):
    m = jnp.max(vals, axis=1, keepdims=True)
    idx = jnp.min(jnp.where(vals == m, iota, E), axis=1, keepdims=True)
    return m, idx


def _gate_body(x_ref, Wg_ref, bg_ref, sval_ref, sidx_ref, aux_ref):
    xs = x_ref[...]
    xf = jnp.mean(xs, axis=(1, 2))
    logits = jax.lax.dot_general(xf, Wg_ref[...], (((1,), (0,)), ((), ())),
                                 precision=_HIGHEST) + bg_ref[...]
    z = logits - jnp.max(logits, axis=1, keepdims=True)
    ez = jnp.exp(z)
    raw = ez / jnp.sum(ez, axis=1, keepdims=True)

    iota = jax.lax.broadcasted_iota(jnp.int32, (B, E), 1)
    _, i1 = _top1(raw, iota)
    mask1 = iota == i1
    _, i2 = _top1(jnp.where(mask1, -1.0, raw), iota)
    mask = jnp.logical_or(mask1, iota == i2).astype(jnp.float32)

    masked = raw * mask
    denom = jnp.sum(masked, axis=0, keepdims=True) + EPS
    gs = masked / denom * CAP

    importance = jnp.mean(raw, axis=0, keepdims=True)
    load = jnp.mean(mask, axis=0, keepdims=True)
    aux_ref[...] = 0.01 * jnp.mean((load - importance) ** 2,
                                   axis=(0, 1), keepdims=True)

    g1v, gi1 = _top1(gs, iota)
    g2v, gi2 = _top1(jnp.where(iota == gi1, -1.0, gs), iota)
    sval_ref[...] = jnp.concatenate([g1v, g2v], axis=1)
    sidx_ref[...] = jnp.concatenate([gi1, gi2], axis=1).astype(jnp.int32)


def _expert_body(sidx_ref, sval_ref, x_ref,
                 Wx0_ref, Wdt0_ref, bdt0_ref, Alog0_ref, Dexp0_ref,
                 g10_ref, b10_ref, g20_ref, b20_ref,
                 Wx1_ref, Wdt1_ref, bdt1_ref, Alog1_ref, Dexp1_ref,
                 g11_ref, b11_ref, g21_ref, b21_ref,
                 out_ref, da_ref, dbu_ref, hsf_ref, hsb_ref):
    b = pl.program_id(0) * (B // 2) + pl.program_id(1)
    Wx = (Wx0_ref, Wx1_ref)
    Wdt = (Wdt0_ref, Wdt1_ref)
    bdt = (bdt0_ref, bdt1_ref)
    Alog = (Alog0_ref, Alog1_ref)
    Dexp = (Dexp0_ref, Dexp1_ref)
    g1 = (g10_ref, g11_ref)
    b1 = (b10_ref, b11_ref)
    g2 = (g20_ref, g21_ref)
    b2 = (b20_ref, b21_ref)

    pio = jax.lax.broadcasted_iota(jnp.int32, (L, L), 0)
    qio = jax.lax.broadcasted_iota(jnp.int32, (L, L), 1)
    T = jnp.logical_and(pio // W == qio % H, pio % W == qio // H).astype(jnp.float32)

    seq_h = x_ref[0].reshape(L, DIM)
    seq_v = jax.lax.dot_general(T, seq_h, (((1,), (0,)), ((), ())),
                                precision=_HIGHEST)
    seq2 = jnp.concatenate([seq_h, seq_v], axis=0)

    Wcat = jnp.concatenate([Wx[0][0], Wx[1][0]], axis=1)
    xd2 = jax.lax.dot_general(seq2, Wcat, (((1,), (0,)), ((), ())),
                              precision=_HIGHEST)

    Cms = []
    for j in range(2):
        xd = xd2[:, j * NXP:(j + 1) * NXP]
        delta2 = _softplus(
            jax.lax.dot_general(xd[:, :DTRANK], Wdt[j][0],
                                (((1,), (0,)), ((), ())),
                                precision=_HIGHEST) + bdt[j][0])
        Bm2 = xd[:, DTRANK:DTRANK + DSTATE]
        Cms.append(xd[:, DTRANK + DSTATE:])
        AT = -jnp.exp(Alog[j][0]).T
        du2 = delta2 * seq2
        for o in range(2):
            Br = Bm2[o * L:(o + 1) * L][:, :, None]
            for c in range(7):
                s = slice(7 * c, 7 * c + 7)
                g = slice(o * L + 7 * c, o * L + 7 * c + 7)
                da_ref[j, o, s] = jnp.exp(delta2[g][:, None, :] * AT[None])
                dbu_ref[j, o, s] = du2[g][:, None, :] * Br[s]

    hf = [[jnp.zeros((DSTATE, DIM), jnp.float32) for _ in range(2)]
          for _ in range(2)]
    hb = [[jnp.zeros((DSTATE, DIM), jnp.float32) for _ in range(2)]
          for _ in range(2)]
    for t in range(L):
        r = L - 1 - t
        for j in range(2):
            for o in range(2):
                hf[j][o] = da_ref[j, o, t] * hf[j][o] + dbu_ref[j, o, t]
                hb[j][o] = da_ref[j, o, r] * hb[j][o] + dbu_ref[j, o, r]
                hsf_ref[j, o, t] = hf[j][o]
                hsb_ref[j, o, r] = hb[j][o]

    outsum = None
    yvs = []
    yhs = []
    for j in range(2):
        Cr = Cms[j][:, :, None]
        yos = []
        for o in range(2):
            chunks = []
            for c in range(7):
                s = slice(7 * c, 7 * c + 7)
                hsum = hsf_ref[j, o, s] + hsb_ref[j, o, s]
                chunks.append(jnp.sum(hsum * Cr[o * L + 7 * c:
                                                o * L + 7 * c + 7], axis=1))
            yos.append(jnp.concatenate(chunks, axis=0))
        yhs.append(yos[0])
        yvs.append(yos[1])

    yv_cat = jnp.concatenate(yvs, axis=1)
    yv_un = jax.lax.dot_general(T, yv_cat, (((1,), (0,)), ((), ())),
                                precision=_HIGHEST)

    for j in range(2):
        y = yhs[j] + yv_un[:, j * DIM:(j + 1) * DIM] \
            + 4.0 * seq_h * Dexp[j][0]
        y = _ln_rows(y, g1[j][0], b1[j][0])
        pooled = jnp.mean(y, axis=0, keepdims=True)
        outv = _ln_rows(pooled, g2[j][0], b2[j][0])
        contrib = sval_ref[b, j] * outv
        outsum = contrib if outsum is None else outsum + contrib

    out_ref[0] = outsum


@jax.jit
def kernel(x, Wg, bg, Wx, Wdt, bdt, A_log, Dexp, g1, b1, g2, b2):
    sval, sidx, aux = pl.pallas_call(
        _gate_body,
        out_shape=[
            jax.ShapeDtypeStruct((B, TOPK), jnp.float32),
            jax.ShapeDtypeStruct((B, TOPK), jnp.int32),
            jax.ShapeDtypeStruct((1, 1), jnp.float32),
        ],
    )(x, Wg, bg.reshape(1, E))

    def expert_specs(j):
        def em3(c, i, si, sv, _j=j):
            return (si[c * (B // 2) + i, _j], 0, 0)
        return [
            pl.BlockSpec((1, DIM, NXP), em3),
            pl.BlockSpec((1, DTRANK, DIM), em3),
            pl.BlockSpec((1, 1, DIM), em3),
            pl.BlockSpec((1, DIM, DSTATE), em3),
            pl.BlockSpec((1, 1, DIM), em3),
            pl.BlockSpec((1, 1, DIM), em3),
            pl.BlockSpec((1, 1, DIM), em3),
            pl.BlockSpec((1, 1, DIM), em3),
            pl.BlockSpec((1, 1, DIM), em3),
        ]

    grid_spec = pltpu.PrefetchScalarGridSpec(
        num_scalar_prefetch=2,
        grid=(2, B // 2),
        in_specs=(
            [pl.BlockSpec((1, H, W, DIM),
                          lambda c, i, si, sv: (c * (B // 2) + i, 0, 0, 0))]
            + expert_specs(0) + expert_specs(1)
        ),
        out_specs=pl.BlockSpec((1, 1, DIM),
                               lambda c, i, si, sv: (c * (B // 2) + i, 0, 0)),
        scratch_shapes=[
            pltpu.VMEM((2, 2, L, DSTATE, DIM), jnp.float32),
            pltpu.VMEM((2, 2, L, DSTATE, DIM), jnp.float32),
            pltpu.VMEM((2, 2, L, DSTATE, DIM), jnp.float32),
            pltpu.VMEM((2, 2, L, DSTATE, DIM), jnp.float32),
        ],
    )

    r3 = lambda a: a.reshape(E, 1, DIM)
    eargs = (Wx, Wdt, r3(bdt), A_log, r3(Dexp), r3(g1), r3(b1), r3(g2), r3(b2))
    mixed = pl.pallas_call(
        _expert_body,
        grid_spec=grid_spec,
        out_shape=jax.ShapeDtypeStruct((B, 1, DIM), jnp.float32),
        compiler_params=pltpu.CompilerParams(
            dimension_semantics=("parallel", "arbitrary"),
        ),
    )(sidx, sval, x, *eargs, *eargs)

    return mixed.reshape(B, DIM), aux[0, 0]
